# Initial kernel scaffold; baseline (speedup 1.0000x reference)
import jax
import jax.numpy as jnp
from jax.experimental import pallas as pl


def kernel(h, x, edge_index, edge_weight, Wp, bp, W0, W1, cheb_b, Wl, Wr, sage_b, W2, b2):
    raise NotImplementedError("write your pallas kernel here")


# speedup vs baseline: 8.1579x; 8.1579x over previous
import functools

import jax
import jax.numpy as jnp
from jax import lax
from jax.experimental import pallas as pl
from jax.experimental.pallas import tpu as pltpu
from jax.experimental.pallas import tpu_sc as plsc

N = 10000
E = 320000
F = 128
HALF = 64
NC = 2
NT = 16
ROWS_PER_TILE = N // NT
CH = 80
EB = 2000


def _leaky(v):
    return jnp.where(v >= 0, v, 0.01 * v)


def _edge_stats_body(row_hbm, col_hbm, ew_hbm, out_hbm,
                     deg_acc, cnt_acc, rowb, colb, ewb,
                     ridx, cidx, degval, onesb, zbuf):
    c = lax.axis_index("c")
    s = lax.axis_index("s")
    e_per_core = E // NC
    e_per_tile = e_per_core // NT
    n_chunks = e_per_tile // CH
    base = c * e_per_core + s * e_per_tile

    zeros16 = jnp.zeros((16,), jnp.float32)
    ones16 = jnp.ones((16,), jnp.float32)

    def init_i(i, _):
        onesb[i, :] = ones16
        return 0
    lax.fori_loop(0, CH, init_i, 0)

    def zrow(i, _):
        zbuf[i, :] = zeros16
        return 0
    lax.fori_loop(0, 200, zrow, 0)

    @pl.when(s < 10)
    def _():
        for q in range(5):
            r0 = s * 1000 + q * 200
            pltpu.sync_copy(zbuf, deg_acc.at[pl.ds(r0, 200)])
            pltpu.sync_copy(zbuf, cnt_acc.at[pl.ds(r0, 200)])
    plsc.subcore_barrier()

    pltpu.sync_copy(row_hbm.at[pl.ds(base, e_per_tile)], rowb)
    pltpu.sync_copy(col_hbm.at[pl.ds(base, e_per_tile)], colb)
    pltpu.sync_copy(ew_hbm.at[pl.ds(base, e_per_tile)], ewb)

    def chunk(k, _):
        off = k * CH
        for j in range(CH // 16):
            r16 = rowb[pl.ds(off + j * 16, 16)]
            c16 = colb[pl.ds(off + j * 16, 16)]
            ridx[pl.ds(j * 16, 16)] = r16
            cidx[pl.ds(j * 16, 16)] = c16

        def bcast(i, _):
            iv = jnp.full((16,), off + i, jnp.int32)
            degval[i, :] = plsc.load_gather(ewb, [iv])
            return 0
        lax.fori_loop(0, CH, bcast, 0)
        pltpu.sync_copy(degval, deg_acc.at[ridx], add=True)
        pltpu.sync_copy(onesb, cnt_acc.at[cidx], add=True)
        return 0
    lax.fori_loop(0, n_chunks, chunk, 0)
    plsc.subcore_barrier()

    @pl.when(s < 10)
    def _():
        for q in range(5):
            r0 = s * 1000 + q * 200
            pltpu.sync_copy(deg_acc.at[pl.ds(r0, 200)], out_hbm.at[c, 0, pl.ds(r0, 200)])
            pltpu.sync_copy(cnt_acc.at[pl.ds(r0, 200)], out_hbm.at[c, 1, pl.ds(r0, 200)])


def _edge_stats(row, col, edge_weight):
    mesh = plsc.VectorSubcoreMesh(core_axis_name="c", subcore_axis_name="s", num_cores=NC, num_subcores=NT)
    f = pl.kernel(
        _edge_stats_body,
        out_type=jax.ShapeDtypeStruct((NC, 2, N, 16), jnp.float32),
        mesh=mesh,
        compiler_params=pltpu.CompilerParams(needs_layout_passes=False, use_tc_tiling_on_sc=False),
        scratch_types=[
            pltpu.VMEM_SHARED((N, 16), jnp.float32),
            pltpu.VMEM_SHARED((N, 16), jnp.float32),
            pltpu.VMEM((E // NC // NT,), jnp.int32),
            pltpu.VMEM((E // NC // NT,), jnp.int32),
            pltpu.VMEM((E // NC // NT,), jnp.float32),
            pltpu.VMEM((CH,), jnp.int32),
            pltpu.VMEM((CH,), jnp.int32),
            pltpu.VMEM((CH, 16), jnp.float32),
            pltpu.VMEM((CH, 16), jnp.float32),
            pltpu.VMEM((200, 16), jnp.float32),
        ],
    )
    return f(row, col, edge_weight)


def _proj_body(x_ref, wp_ref, bp_ref, dc_ref, xp2_ref, dinv_ref):
    xp = jnp.dot(x_ref[...], wp_ref[...], preferred_element_type=jnp.float32)
    xp = xp + bp_ref[...]
    xp2_ref[0] = xp[:, :HALF]
    xp2_ref[1] = xp[:, HALF:]
    deg = dc_ref[0, 0, :, 0:1] + dc_ref[1, 0, :, 0:1]
    dinv_ref[...] = jnp.where(deg > 0, lax.rsqrt(jnp.where(deg > 0, deg, 1.0)), 0.0)


def _proj(x, Wp, bp, degcnt):
    R = 400
    grid = (N // R,)
    return pl.pallas_call(
        _proj_body,
        grid=grid,
        in_specs=[
            pl.BlockSpec((R, F), lambda i: (i, 0)),
            pl.BlockSpec((F, F), lambda i: (0, 0)),
            pl.BlockSpec((1, F), lambda i: (0, 0)),
            pl.BlockSpec((NC, 2, R, 16), lambda i: (0, 0, i, 0)),
        ],
        out_specs=[
            pl.BlockSpec((NC, R, HALF), lambda i: (0, i, 0)),
            pl.BlockSpec((R, 1), lambda i: (i, 0)),
        ],
        out_shape=[
            jax.ShapeDtypeStruct((NC, N, HALF), jnp.float32),
            jax.ShapeDtypeStruct((N, 1), jnp.float32),
        ],
    )(x, Wp, bp, degcnt)


def _edge_agg_body(row_hbm, col_hbm, ew_hbm, dinv_hbm, xp2_hbm, tx_hbm, ns_hbm,
                   tx_acc, ns_acc, dinvb, rowb, colb, ewb,
                   rows_b, wrows, gbuf, cidx, wbuf, zbuf, sem):
    c = lax.axis_index("c")
    s = lax.axis_index("s")
    e_per_tile = E // NT
    base = s * e_per_tile
    tab_base = c * N

    zeros16 = jnp.zeros((16,), jnp.float32)

    def zrow(i, _):
        for q in range(HALF // 16):
            zbuf[i, pl.ds(q * 16, 16)] = zeros16
        return 0
    lax.fori_loop(0, 200, zrow, 0)

    @pl.when(s < 10)
    def _():
        for q in range(5):
            r0 = s * 1000 + q * 200
            pltpu.sync_copy(zbuf, tx_acc.at[pl.ds(r0, 200)])
            pltpu.sync_copy(zbuf, ns_acc.at[pl.ds(r0, 200)])
    plsc.subcore_barrier()

    pltpu.sync_copy(dinv_hbm, dinvb)

    def block(b, _):
        bb = base + b * EB
        pltpu.sync_copy(row_hbm.at[pl.ds(bb, EB)], rowb)
        pltpu.sync_copy(col_hbm.at[pl.ds(bb, EB)], colb)
        pltpu.sync_copy(ew_hbm.at[pl.ds(bb, EB)], ewb)
        lax.fori_loop(0, EB // CH, chunk, 0)
        return 0

    def chunk(k, _):
        off = k * CH
        for j in range(CH // 16):
            r16 = rowb[pl.ds(off + j * 16, 16)]
            c16 = colb[pl.ds(off + j * 16, 16)]
            e16 = ewb[pl.ds(off + j * 16, 16)]
            gbuf[pl.ds(j * 16, 16)] = r16 + tab_base
            cidx[pl.ds(j * 16, 16)] = c16
            dr = plsc.load_gather(dinvb, [r16])
            dc = plsc.load_gather(dinvb, [c16])
            w16 = -(dr * e16 * dc)
            w16 = jnp.where(r16 == c16, w16 - 1.0, w16)
            wbuf[pl.ds(j * 16, 16)] = w16
        pltpu.async_copy(xp2_hbm.at[gbuf], rows_b, sem).wait()

        def wmul(i, _):
            wv = plsc.load_gather(wbuf, [jnp.full((16,), i, jnp.int32)])
            for q in range(HALF // 16):
                wrows[i, pl.ds(q * 16, 16)] = rows_b[i, pl.ds(q * 16, 16)] * wv
            return 0
        lax.fori_loop(0, CH, wmul, 0)
        pltpu.sync_copy(wrows, tx_acc.at[cidx], add=True)
        pltpu.sync_copy(rows_b, ns_acc.at[cidx], add=True)
        return 0
    lax.fori_loop(0, e_per_tile // EB, block, 0)
    plsc.subcore_barrier()

    @pl.when(s < 10)
    def _():
        for q in range(5):
            r0 = s * 1000 + q * 200
            pltpu.sync_copy(tx_acc.at[pl.ds(r0, 200)], tx_hbm.at[c, pl.ds(r0, 200)])
            pltpu.sync_copy(ns_acc.at[pl.ds(r0, 200)], ns_hbm.at[c, pl.ds(r0, 200)])


def _edge_agg(row, col, edge_weight, dinv, xp2_flat):
    mesh = plsc.VectorSubcoreMesh(core_axis_name="c", subcore_axis_name="s", num_cores=NC, num_subcores=NT)
    f = pl.kernel(
        _edge_agg_body,
        out_type=(
            jax.ShapeDtypeStruct((NC, N, HALF), jnp.float32),
            jax.ShapeDtypeStruct((NC, N, HALF), jnp.float32),
        ),
        mesh=mesh,
        compiler_params=pltpu.CompilerParams(needs_layout_passes=False, use_tc_tiling_on_sc=False),
        scratch_types=[
            pltpu.VMEM_SHARED((N, HALF), jnp.float32),
            pltpu.VMEM_SHARED((N, HALF), jnp.float32),
            pltpu.VMEM((N,), jnp.float32),
            pltpu.VMEM((EB,), jnp.int32),
            pltpu.VMEM((EB,), jnp.int32),
            pltpu.VMEM((EB,), jnp.float32),
            pltpu.VMEM((CH, HALF), jnp.float32),
            pltpu.VMEM((CH, HALF), jnp.float32),
            pltpu.VMEM((CH,), jnp.int32),
            pltpu.VMEM((CH,), jnp.int32),
            pltpu.VMEM((CH,), jnp.float32),
            pltpu.VMEM((200, HALF), jnp.float32),
            pltpu.SemaphoreType.DMA,
        ],
    )
    return f(row, col, edge_weight, dinv, xp2_flat)


def _head_body(xp2_ref, tx_ref, ns_ref, dc_ref,
               w0_ref, w1_ref, cb_ref, wl_ref, wr_ref, sb_ref, w2_ref, b2_ref,
               out_ref):
    xp = jnp.concatenate([xp2_ref[0], xp2_ref[1]], axis=1)
    tx1 = jnp.concatenate([tx_ref[0], tx_ref[1]], axis=1)
    ns = jnp.concatenate([ns_ref[0], ns_ref[1]], axis=1)
    cnt = dc_ref[0, 1, :, 0:1] + dc_ref[1, 1, :, 0:1]
    mean = ns / jnp.maximum(cnt, 1.0)
    dot = functools.partial(jnp.dot, preferred_element_type=jnp.float32)
    o1 = _leaky(dot(xp, w0_ref[...]) + dot(tx1, w1_ref[...]) + cb_ref[...])
    o2 = _leaky(dot(mean, wl_ref[...]) + dot(xp, wr_ref[...]) + sb_ref[...])
    out_ref[...] = dot(o1 + o2, w2_ref[...]) + b2_ref[...]


def _head(xp2, tx, ns, degcnt, W0, W1, cheb_b, Wl, Wr, sage_b, W2, b2):
    R = 400
    grid = (N // R,)
    mat = pl.BlockSpec((F, F), lambda i: (0, 0))
    vec = pl.BlockSpec((1, F), lambda i: (0, 0))
    half3 = pl.BlockSpec((NC, R, HALF), lambda i: (0, i, 0))
    return pl.pallas_call(
        _head_body,
        grid=grid,
        in_specs=[half3, half3, half3,
                  pl.BlockSpec((NC, 2, R, 16), lambda i: (0, 0, i, 0)),
                  mat, mat, vec, mat, mat, vec, mat, vec],
        out_specs=pl.BlockSpec((R, F), lambda i: (i, 0)),
        out_shape=jax.ShapeDtypeStruct((N, F), jnp.float32),
    )(xp2, tx, ns, degcnt, W0, W1, cheb_b, Wl, Wr, sage_b, W2, b2)


def kernel(h, x, edge_index, edge_weight, Wp, bp, W0, W1, cheb_b, Wl, Wr, sage_b, W2, b2):
    row = edge_index[0]
    col = edge_index[1]
    degcnt = _edge_stats(row, col, edge_weight)
    xp2, dinv = _proj(x, Wp, bp.reshape(1, F), degcnt)
    tx, ns = _edge_agg(row, col, edge_weight, dinv.reshape(N),
                       xp2.reshape(NC * N, HALF))
    o3 = _head(xp2, tx, ns, degcnt, W0, W1, cheb_b.reshape(1, F),
               Wl, Wr, sage_b.reshape(1, F), W2, b2.reshape(1, F))
    return (x, o3)


# speedup vs baseline: 11.7026x; 1.4345x over previous
import functools

import jax
import jax.numpy as jnp
from jax import lax
from jax.experimental import pallas as pl
from jax.experimental.pallas import tpu as pltpu
from jax.experimental.pallas import tpu_sc as plsc

N = 10000
E = 320000
F = 128
HALF = 64
NC = 2
NT = 16
ROWS_PER_TILE = N // NT
CH = 80
EB = 2000


def _leaky(v):
    return jnp.where(v >= 0, v, 0.01 * v)


def _edge_stats_body(row_hbm, col_hbm, ew_hbm, out_hbm,
                     deg_acc, cnt_acc, rowb, colb, ewb,
                     ridx, cidx, degval, onesb, zbuf):
    c = lax.axis_index("c")
    s = lax.axis_index("s")
    e_per_core = E // NC
    e_per_tile = e_per_core // NT
    n_chunks = e_per_tile // CH
    base = c * e_per_core + s * e_per_tile

    zeros16 = jnp.zeros((16,), jnp.float32)
    ones16 = jnp.ones((16,), jnp.float32)

    def init_i(i, _):
        onesb[i, :] = ones16
        return 0
    lax.fori_loop(0, CH, init_i, 0)

    def zrow(i, _):
        zbuf[i, :] = zeros16
        return 0
    lax.fori_loop(0, 200, zrow, 0)

    @pl.when(s < 10)
    def _():
        for q in range(5):
            r0 = s * 1000 + q * 200
            pltpu.sync_copy(zbuf, deg_acc.at[pl.ds(r0, 200)])
            pltpu.sync_copy(zbuf, cnt_acc.at[pl.ds(r0, 200)])
    plsc.subcore_barrier()

    pltpu.sync_copy(row_hbm.at[pl.ds(base, e_per_tile)], rowb)
    pltpu.sync_copy(col_hbm.at[pl.ds(base, e_per_tile)], colb)
    pltpu.sync_copy(ew_hbm.at[pl.ds(base, e_per_tile)], ewb)

    def chunk(k, _):
        off = k * CH
        for j in range(CH // 16):
            r16 = rowb[pl.ds(off + j * 16, 16)]
            c16 = colb[pl.ds(off + j * 16, 16)]
            ridx[pl.ds(j * 16, 16)] = r16
            cidx[pl.ds(j * 16, 16)] = c16

        def bcast(i, _):
            iv = jnp.full((16,), off + i, jnp.int32)
            degval[i, :] = plsc.load_gather(ewb, [iv])
            return 0
        lax.fori_loop(0, CH, bcast, 0)
        pltpu.sync_copy(degval, deg_acc.at[ridx], add=True)
        pltpu.sync_copy(onesb, cnt_acc.at[cidx], add=True)
        return 0
    lax.fori_loop(0, n_chunks, chunk, 0)
    plsc.subcore_barrier()

    @pl.when(s < 10)
    def _():
        for q in range(5):
            r0 = s * 1000 + q * 200
            pltpu.sync_copy(deg_acc.at[pl.ds(r0, 200)], out_hbm.at[c, 0, pl.ds(r0, 200)])
            pltpu.sync_copy(cnt_acc.at[pl.ds(r0, 200)], out_hbm.at[c, 1, pl.ds(r0, 200)])


def _edge_stats(row, col, edge_weight):
    mesh = plsc.VectorSubcoreMesh(core_axis_name="c", subcore_axis_name="s", num_cores=NC, num_subcores=NT)
    f = pl.kernel(
        _edge_stats_body,
        out_type=jax.ShapeDtypeStruct((NC, 2, N, 16), jnp.float32),
        mesh=mesh,
        compiler_params=pltpu.CompilerParams(needs_layout_passes=False, use_tc_tiling_on_sc=False),
        scratch_types=[
            pltpu.VMEM_SHARED((N, 16), jnp.float32),
            pltpu.VMEM_SHARED((N, 16), jnp.float32),
            pltpu.VMEM((E // NC // NT,), jnp.int32),
            pltpu.VMEM((E // NC // NT,), jnp.int32),
            pltpu.VMEM((E // NC // NT,), jnp.float32),
            pltpu.VMEM((CH,), jnp.int32),
            pltpu.VMEM((CH,), jnp.int32),
            pltpu.VMEM((CH, 16), jnp.float32),
            pltpu.VMEM((CH, 16), jnp.float32),
            pltpu.VMEM((200, 16), jnp.float32),
        ],
    )
    return f(row, col, edge_weight)


def _proj_body(x_ref, wp_ref, bp_ref, dc_ref, xp2_ref, dinv_ref):
    xp = jnp.dot(x_ref[...], wp_ref[...], preferred_element_type=jnp.float32)
    xp = xp + bp_ref[...]
    xp2_ref[0] = xp[:, :HALF]
    xp2_ref[1] = xp[:, HALF:]
    deg = dc_ref[0, 0, :, 0:1] + dc_ref[1, 0, :, 0:1]
    dinv_ref[...] = jnp.where(deg > 0, lax.rsqrt(jnp.where(deg > 0, deg, 1.0)), 0.0)


def _proj(x, Wp, bp, degcnt):
    R = 400
    grid = (N // R,)
    return pl.pallas_call(
        _proj_body,
        grid=grid,
        in_specs=[
            pl.BlockSpec((R, F), lambda i: (i, 0)),
            pl.BlockSpec((F, F), lambda i: (0, 0)),
            pl.BlockSpec((1, F), lambda i: (0, 0)),
            pl.BlockSpec((NC, 2, R, 16), lambda i: (0, 0, i, 0)),
        ],
        out_specs=[
            pl.BlockSpec((NC, R, HALF), lambda i: (0, i, 0)),
            pl.BlockSpec((R, 1), lambda i: (i, 0)),
        ],
        out_shape=[
            jax.ShapeDtypeStruct((NC, N, HALF), jnp.float32),
            jax.ShapeDtypeStruct((N, 1), jnp.float32),
        ],
    )(x, Wp, bp, degcnt)


def _edge_agg_body(row_hbm, col_hbm, ew_hbm, dinv_hbm, xp2_hbm, tx_hbm, ns_hbm,
                   tx_acc, ns_acc, dinvb, rowb, colb, ewb,
                   r0b, r1b, r2b, w0b, w1b, w2b,
                   g0b, g1b, g2b, c0b, c1b, c2b, wbuf,
                   gs0, gs1, gs2, ss0, ss1, ss2):
    c = lax.axis_index("c")
    s = lax.axis_index("s")
    e_per_tile = E // NT
    base = s * e_per_tile
    tab_base = c * N
    rows_b = [r0b, r1b, r2b]
    wrows = [w0b, w1b, w2b]
    gbuf = [g0b, g1b, g2b]
    cidx = [c0b, c1b, c2b]
    gsem = [gs0, gs1, gs2]
    ssem = [ss0, ss1, ss2]
    KPB = EB // CH

    zeros16 = jnp.zeros((16,), jnp.float32)

    def zrow(i, _):
        for q in range(HALF // 16):
            w0b[i, pl.ds(q * 16, 16)] = zeros16
        return 0
    lax.fori_loop(0, CH, zrow, 0)

    @pl.when(s < 10)
    def _():
        def zcp(q, _):
            r0 = s * 1000 + q * 40
            pltpu.sync_copy(w0b.at[pl.ds(0, 40)], tx_acc.at[pl.ds(r0, 40)])
            pltpu.sync_copy(w0b.at[pl.ds(0, 40)], ns_acc.at[pl.ds(r0, 40)])
            return 0
        lax.fori_loop(0, 25, zcp, 0)
    plsc.subcore_barrier()

    pltpu.sync_copy(dinv_hbm, dinvb)

    def build(k, sl):
        off = k * CH
        for j in range(CH // 16):
            r16 = rowb[pl.ds(off + j * 16, 16)]
            c16 = colb[pl.ds(off + j * 16, 16)]
            gbuf[sl][pl.ds(j * 16, 16)] = r16 + tab_base
            cidx[sl][pl.ds(j * 16, 16)] = c16

    def compute_w(k):
        off = k * CH
        for j in range(CH // 16):
            r16 = rowb[pl.ds(off + j * 16, 16)]
            c16 = colb[pl.ds(off + j * 16, 16)]
            e16 = ewb[pl.ds(off + j * 16, 16)]
            dr = plsc.load_gather(dinvb, [r16])
            dc = plsc.load_gather(dinvb, [c16])
            w16 = -(dr * e16 * dc)
            w16 = jnp.where(r16 == c16, w16 - 1.0, w16)
            wbuf[pl.ds(j * 16, 16)] = w16

    def wait_scatter(sl):
        pltpu.make_async_copy(wrows[sl], tx_acc.at[cidx[sl]], ssem[sl]).wait()
        pltpu.make_async_copy(rows_b[sl], ns_acc.at[cidx[sl]], ssem[sl]).wait()

    def block(b, _):
        bb = base + b * EB
        pltpu.sync_copy(row_hbm.at[pl.ds(bb, EB)], rowb)
        pltpu.sync_copy(col_hbm.at[pl.ds(bb, EB)], colb)
        pltpu.sync_copy(ew_hbm.at[pl.ds(bb, EB)], ewb)

        build(0, 0)
        pltpu.async_copy(xp2_hbm.at[gbuf[0]], rows_b[0], gsem[0])
        for k in range(KPB):
            p = k % 3
            n = (k + 1) % 3
            if k + 1 < KPB:
                if k - 2 >= 0:
                    wait_scatter(n)
                build(k + 1, n)
                pltpu.async_copy(xp2_hbm.at[gbuf[n]], rows_b[n], gsem[n])
            pltpu.make_async_copy(xp2_hbm.at[gbuf[p]], rows_b[p], gsem[p]).wait()
            compute_w(k)

            def wmul(i, _):
                wv = plsc.load_gather(wbuf, [jnp.full((16,), i, jnp.int32)])
                for q in range(HALF // 16):
                    wrows[p][i, pl.ds(q * 16, 16)] = rows_b[p][i, pl.ds(q * 16, 16)] * wv
                return 0
            lax.fori_loop(0, CH, wmul, 0)
            pltpu.async_copy(wrows[p], tx_acc.at[cidx[p]], ssem[p], add=True)
            pltpu.async_copy(rows_b[p], ns_acc.at[cidx[p]], ssem[p], add=True)
        for sl in (1, 2, 0):
            wait_scatter(sl)
        return 0
    lax.fori_loop(0, e_per_tile // EB, block, 0)
    plsc.subcore_barrier()

    @pl.when(s < 10)
    def _():
        for q in range(5):
            r0 = s * 1000 + q * 200
            pltpu.sync_copy(tx_acc.at[pl.ds(r0, 200)], tx_hbm.at[c, pl.ds(r0, 200)])
            pltpu.sync_copy(ns_acc.at[pl.ds(r0, 200)], ns_hbm.at[c, pl.ds(r0, 200)])


def _edge_agg(row, col, edge_weight, dinv, xp2_flat):
    mesh = plsc.VectorSubcoreMesh(core_axis_name="c", subcore_axis_name="s", num_cores=NC, num_subcores=NT)
    f = pl.kernel(
        _edge_agg_body,
        out_type=(
            jax.ShapeDtypeStruct((NC, N, HALF), jnp.float32),
            jax.ShapeDtypeStruct((NC, N, HALF), jnp.float32),
        ),
        mesh=mesh,
        compiler_params=pltpu.CompilerParams(needs_layout_passes=False, use_tc_tiling_on_sc=False),
        scratch_types=[
            pltpu.VMEM_SHARED((N, HALF), jnp.float32),
            pltpu.VMEM_SHARED((N, HALF), jnp.float32),
            pltpu.VMEM((N,), jnp.float32),
            pltpu.VMEM((EB,), jnp.int32),
            pltpu.VMEM((EB,), jnp.int32),
            pltpu.VMEM((EB,), jnp.float32),
            pltpu.VMEM((CH, HALF), jnp.float32),
            pltpu.VMEM((CH, HALF), jnp.float32),
            pltpu.VMEM((CH, HALF), jnp.float32),
            pltpu.VMEM((CH, HALF), jnp.float32),
            pltpu.VMEM((CH, HALF), jnp.float32),
            pltpu.VMEM((CH, HALF), jnp.float32),
            pltpu.VMEM((CH,), jnp.int32),
            pltpu.VMEM((CH,), jnp.int32),
            pltpu.VMEM((CH,), jnp.int32),
            pltpu.VMEM((CH,), jnp.int32),
            pltpu.VMEM((CH,), jnp.int32),
            pltpu.VMEM((CH,), jnp.int32),
            pltpu.VMEM((CH,), jnp.float32),
            pltpu.SemaphoreType.DMA,
            pltpu.SemaphoreType.DMA,
            pltpu.SemaphoreType.DMA,
            pltpu.SemaphoreType.DMA,
            pltpu.SemaphoreType.DMA,
            pltpu.SemaphoreType.DMA,
        ],
    )
    return f(row, col, edge_weight, dinv, xp2_flat)


def _head_body(xp2_ref, tx_ref, ns_ref, dc_ref,
               w0_ref, w1_ref, cb_ref, wl_ref, wr_ref, sb_ref, w2_ref, b2_ref,
               out_ref):
    xp = jnp.concatenate([xp2_ref[0], xp2_ref[1]], axis=1)
    tx1 = jnp.concatenate([tx_ref[0], tx_ref[1]], axis=1)
    ns = jnp.concatenate([ns_ref[0], ns_ref[1]], axis=1)
    cnt = dc_ref[0, 1, :, 0:1] + dc_ref[1, 1, :, 0:1]
    mean = ns / jnp.maximum(cnt, 1.0)
    dot = functools.partial(jnp.dot, preferred_element_type=jnp.float32)
    o1 = _leaky(dot(xp, w0_ref[...]) + dot(tx1, w1_ref[...]) + cb_ref[...])
    o2 = _leaky(dot(mean, wl_ref[...]) + dot(xp, wr_ref[...]) + sb_ref[...])
    out_ref[...] = dot(o1 + o2, w2_ref[...]) + b2_ref[...]


def _head(xp2, tx, ns, degcnt, W0, W1, cheb_b, Wl, Wr, sage_b, W2, b2):
    R = 400
    grid = (N // R,)
    mat = pl.BlockSpec((F, F), lambda i: (0, 0))
    vec = pl.BlockSpec((1, F), lambda i: (0, 0))
    half3 = pl.BlockSpec((NC, R, HALF), lambda i: (0, i, 0))
    return pl.pallas_call(
        _head_body,
        grid=grid,
        in_specs=[half3, half3, half3,
                  pl.BlockSpec((NC, 2, R, 16), lambda i: (0, 0, i, 0)),
                  mat, mat, vec, mat, mat, vec, mat, vec],
        out_specs=pl.BlockSpec((R, F), lambda i: (i, 0)),
        out_shape=jax.ShapeDtypeStruct((N, F), jnp.float32),
    )(xp2, tx, ns, degcnt, W0, W1, cheb_b, Wl, Wr, sage_b, W2, b2)


def kernel(h, x, edge_index, edge_weight, Wp, bp, W0, W1, cheb_b, Wl, Wr, sage_b, W2, b2):
    row = edge_index[0]
    col = edge_index[1]
    degcnt = _edge_stats(row, col, edge_weight)
    xp2, dinv = _proj(x, Wp, bp.reshape(1, F), degcnt)
    tx, ns = _edge_agg(row, col, edge_weight, dinv.reshape(N),
                       xp2.reshape(NC * N, HALF))
    o3 = _head(xp2, tx, ns, degcnt, W0, W1, cheb_b.reshape(1, F),
               Wl, Wr, sage_b.reshape(1, F), W2, b2.reshape(1, F))
    return (x, o3)
